# baseline (speedup 1.0000x reference)
import jax
import jax.numpy as jnp
from jax import lax
from jax.experimental import pallas as pl
from jax.experimental.pallas import tpu as pltpu
from jax.experimental.pallas import tpu_sc as plsc

U = 512
B = 128
D = 8
C = 2
K = 8192
W = D * U
OUT = B * U
ST = (D - 1) * U
FULL = OUT + ST
NK = FULL // U
NCH = C * NK
L = 16
NG = U // L

_NC = 2
_NS = 16
_NW = _NC * _NS
_CPW = -(-NCH // _NW)


def _sc_body(upd_hbm, st_hbm, out0_hbm, out1_hbm, buf, sbuf, obuf,
             sem0, sem1):
    sems = (sem0, sem1)
    wid = lax.axis_index("s") * _NC + lax.axis_index("c")

    def issue(ch, slot):
        @pl.when(ch < NCH)
        def _():
            c = ch // NK
            k = ch % NK
            for d in range(D):
                b = jnp.clip(k - d, 0, B - 1)
                pltpu.make_async_copy(
                    upd_hbm.at[b, c, pl.ds(W + d * U, U)],
                    buf.at[slot, d], sems[slot],
                ).start()
            sk = jnp.minimum(k, D - 2)
            pltpu.make_async_copy(
                st_hbm.at[c, pl.ds(sk * U, U)], sbuf.at[slot], sems[slot]
            ).start()

    def process(ch, slot):
        @pl.when(ch < NCH)
        def _():
            c = ch // NK
            k = ch % NK

            for d in range(D):
                pltpu.make_async_copy(
                    upd_hbm.at[0, 0, pl.ds(0, U)], buf.at[slot, d],
                    sems[slot],
                ).wait()
            pltpu.make_async_copy(
                st_hbm.at[0, pl.ds(0, U)], sbuf.at[slot], sems[slot]
            ).wait()

            z = jnp.zeros((L,), jnp.float32)
            for d in range(D):
                @pl.when((k - d < 0) | (k - d > B - 1))
                def _(d=d):
                    for i in range(NG):
                        buf[slot, d, pl.ds(i * L, L)] = z

            gate = jnp.where(k < D - 1, jnp.float32(1.0), jnp.float32(0.0))
            for i in range(NG):
                g = pl.ds(i * L, L)
                s01 = buf[slot, 0, g] + buf[slot, 1, g]
                s23 = buf[slot, 2, g] + buf[slot, 3, g]
                s45 = buf[slot, 4, g] + buf[slot, 5, g]
                s67 = buf[slot, 6, g] + buf[slot, 7, g]
                s = (s01 + s23) + (s45 + s67)
                obuf[g] = s * jnp.float32(1.0 / D) + sbuf[slot, g] * gate

            @pl.when(k < B)
            def _():
                pltpu.sync_copy(obuf, out0_hbm.at[c, pl.ds(k * U, U)])

            @pl.when(k >= B)
            def _():
                pltpu.sync_copy(obuf, out1_hbm.at[c, pl.ds((k - B) * U, U)])

    def ch_of(j):
        return j * _NW + wid

    issue(ch_of(0), 0)

    def pair(t, carry):
        j0 = 2 * t
        issue(ch_of(j0 + 1), 1)
        process(ch_of(j0), 0)
        issue(ch_of(j0 + 2), 0)
        process(ch_of(j0 + 1), 1)
        return carry

    lax.fori_loop(0, (_CPW + 1) // 2, pair, 0)


@jax.jit
def kernel(update, state):
    mesh = plsc.VectorSubcoreMesh(core_axis_name="c", subcore_axis_name="s")
    return pl.kernel(
        _sc_body,
        out_type=(
            jax.ShapeDtypeStruct((C, OUT), jnp.float32),
            jax.ShapeDtypeStruct((C, ST), jnp.float32),
        ),
        mesh=mesh,
        scratch_types=[
            pltpu.VMEM((2, D, U), jnp.float32),
            pltpu.VMEM((2, U), jnp.float32),
            pltpu.VMEM((U,), jnp.float32),
            pltpu.SemaphoreType.DMA,
            pltpu.SemaphoreType.DMA,
        ],
    )(update, state)
